# baseline (speedup 1.0000x reference)
import jax
import jax.numpy as jnp
from jax import lax
from jax.experimental import pallas as pl
from jax.experimental.pallas import tpu as pltpu
from jax.experimental.pallas import tpu_sc as plsc

N = 10000
D = 128
Q = 32
NQ = D // Q
E = 320000
L = 4

NC = 2
NS = 16

CH = 320
EPW = 20480
NCH = EPW // CH
EPAD = EPW * NS
NP = 10240
ZR = NP // NS


def _dprop_body(sidx_hbm, didx_hbm, xs_hbm, zeros_hbm, y1_hbm, y2_hbm,
                idx, r0, r1, r2, r3, shr,
                gs0, gs1, gs2, gs3, ss0, ss1, ss2, ss3):
    c = lax.axis_index("c")
    s = lax.axis_index("s")
    buf0 = shr.at[0]
    buf1 = shr.at[1]
    pltpu.sync_copy(sidx_hbm.at[s], idx.at[pl.ds(0, NCH)])
    pltpu.sync_copy(didx_hbm.at[s], idx.at[pl.ds(NCH, NCH)])

    def run_pass(staged, acc):
        def gather(row, buf, sem):
            pltpu.async_copy(staged.at[idx.at[row]], buf, sem)

        def wait_gather(buf, sem):
            pltpu.make_async_copy(staged.at[idx.at[0]], buf, sem).wait()

        def scatter(buf, row, sem):
            pltpu.async_copy(buf, acc.at[idx.at[NCH + row]], sem, add=True)

        def wait_scatter(buf, sem):
            pltpu.make_async_copy(buf, acc.at[idx.at[NCH]], sem).wait()

        bufs = [r0, r1, r2, r3]
        gsems = [gs0, gs1, gs2, gs3]
        ssems = [ss0, ss1, ss2, ss3]
        gather(0, r0, gs0)
        gather(1, r1, gs1)

        def step(i, carry):
            for j in range(4):
                c4 = 4 * i + j
                wait_gather(bufs[j], gsems[j])
                scatter(bufs[j], c4, ssems[j])
                jn = (j + 2) % 4
                if j < 2:
                    @pl.when(i > 0)
                    def _():
                        wait_scatter(bufs[jn], ssems[jn])

                    gather(c4 + 2, bufs[jn], gsems[jn])
                else:
                    wait_scatter(bufs[jn], ssems[jn])

                    @pl.when(i < NCH // 4 - 1)
                    def _():
                        gather(c4 + 2, bufs[jn], gsems[jn])

            return carry

        lax.fori_loop(0, NCH // 4, step, 0)
        wait_scatter(r2, ss2)
        wait_scatter(r3, ss3)
        plsc.subcore_barrier()

    rows = pl.ds(s * ZR, ZR)
    for qq in range(NQ // NC):
        q = c * (NQ // NC) + qq
        cols = pl.ds(q * Q, Q)
        pltpu.sync_copy(xs_hbm.at[rows, cols], buf0.at[rows])
        pltpu.sync_copy(zeros_hbm, buf1.at[rows])
        plsc.subcore_barrier()
        run_pass(buf0, buf1)
        pltpu.sync_copy(buf1.at[rows], y1_hbm.at[rows, cols])
        pltpu.sync_copy(zeros_hbm, buf0.at[rows])
        plsc.subcore_barrier()
        run_pass(buf1, buf0)
        pltpu.sync_copy(buf0.at[rows], y2_hbm.at[rows, cols])
        plsc.subcore_barrier()


_dprop = pl.kernel(
    _dprop_body,
    out_type=(jax.ShapeDtypeStruct((NP, D), jnp.float32),
              jax.ShapeDtypeStruct((NP, D), jnp.float32)),
    mesh=plsc.VectorSubcoreMesh(core_axis_name="c", subcore_axis_name="s",
                                num_cores=NC, num_subcores=NS),
    scratch_types=[
        pltpu.VMEM((2 * NCH, CH), jnp.int32),
        pltpu.VMEM((CH, Q), jnp.float32),
        pltpu.VMEM((CH, Q), jnp.float32),
        pltpu.VMEM((CH, Q), jnp.float32),
        pltpu.VMEM((CH, Q), jnp.float32),
        pltpu.VMEM_SHARED((2, NP, Q), jnp.float32),
        pltpu.SemaphoreType.DMA,
        pltpu.SemaphoreType.DMA,
        pltpu.SemaphoreType.DMA,
        pltpu.SemaphoreType.DMA,
        pltpu.SemaphoreType.DMA,
        pltpu.SemaphoreType.DMA,
        pltpu.SemaphoreType.DMA,
        pltpu.SemaphoreType.DMA,
    ],
    compiler_params=pltpu.CompilerParams(use_tc_tiling_on_sc=False),
)


def _embed_body(h_ref, w_ref, b_ref, o_ref):
    x = (jnp.dot(h_ref[...], w_ref[...],
                 preferred_element_type=jnp.float32) + b_ref[...])
    o_ref[:N] = x
    o_ref[N:] = jnp.zeros((NP - N, D), jnp.float32)


_embed = pl.pallas_call(
    _embed_body,
    out_shape=jax.ShapeDtypeStruct((NP, D), jnp.float32),
)


def _tail_body(x_ref, y1_ref, y2_ref, w_ref, b_ref, g_ref, bt_ref, o_ref):
    t = (jnp.dot(x_ref[:N], w_ref[0], preferred_element_type=jnp.float32)
         + jnp.dot(y1_ref[:N], w_ref[1], preferred_element_type=jnp.float32)
         + jnp.dot(y2_ref[:N], w_ref[2], preferred_element_type=jnp.float32)
         + b_ref[...])
    mu = jnp.mean(t, axis=0, keepdims=True)
    var = jnp.mean((t - mu) * (t - mu), axis=0, keepdims=True)
    t = (t - mu) * lax.rsqrt(var + 1e-5) * g_ref[...] + bt_ref[...]
    o_ref[:N] = jnp.maximum(t, 0.0) + x_ref[:N]
    o_ref[N:] = jnp.zeros((NP - N, D), jnp.float32)


_tail = pl.pallas_call(
    _tail_body,
    out_shape=jax.ShapeDtypeStruct((NP, D), jnp.float32),
)


def kernel(h, e, edge_index, W_emb, b_emb, Wl, bl, gamma, beta):
    src = edge_index[0]
    dst = edge_index[1]
    pad = EPAD - E
    src_p = jnp.concatenate([src, jnp.zeros((pad,), jnp.int32)])
    dst_p = jnp.concatenate([dst, jnp.full((pad,), N, jnp.int32)])
    sidxm = src_p.reshape(NS, NCH, CH)
    didxm = dst_p.reshape(NS, NCH, CH)
    zeros = jnp.zeros((ZR, Q), jnp.float32)

    x = _embed(h, W_emb, b_emb.reshape(1, D))
    for l in range(L):
        y1, y2 = _dprop(sidxm, didxm, x, zeros)
        x = _tail(x, y1, y2, Wl[l],
                  (bl[l, 0] + bl[l, 1] + bl[l, 2]).reshape(1, D),
                  gamma[l].reshape(1, D), beta[l].reshape(1, D))
    return x[:N]


# speedup vs baseline: 1.0004x; 1.0004x over previous
import jax
import jax.numpy as jnp
from jax import lax
from jax.experimental import pallas as pl
from jax.experimental.pallas import tpu as pltpu
from jax.experimental.pallas import tpu_sc as plsc

N = 10000
D = 128
Q = 32
NQ = D // Q
E = 320000
L = 4

NC = 2
NS = 16

CH = 256
EPW = 20480
NCH = EPW // CH
EPAD = EPW * NS
NP = 10240
ZR = NP // NS


def _dprop_body(sidx_hbm, didx_hbm, xs_hbm, zeros_hbm, y1_hbm, y2_hbm,
                idx, r0, r1, r2, r3, shr,
                gs0, gs1, gs2, gs3, ss0, ss1, ss2, ss3):
    c = lax.axis_index("c")
    s = lax.axis_index("s")
    buf0 = shr.at[0]
    buf1 = shr.at[1]
    pltpu.sync_copy(sidx_hbm.at[s], idx.at[pl.ds(0, NCH)])
    pltpu.sync_copy(didx_hbm.at[s], idx.at[pl.ds(NCH, NCH)])

    def run_pass(staged, acc):
        def gather(row, buf, sem):
            pltpu.async_copy(staged.at[idx.at[row]], buf, sem)

        def wait_gather(buf, sem):
            pltpu.make_async_copy(staged.at[idx.at[0]], buf, sem).wait()

        def scatter(buf, row, sem):
            pltpu.async_copy(buf, acc.at[idx.at[NCH + row]], sem, add=True)

        def wait_scatter(buf, sem):
            pltpu.make_async_copy(buf, acc.at[idx.at[NCH]], sem).wait()

        bufs = [r0, r1, r2, r3]
        gsems = [gs0, gs1, gs2, gs3]
        ssems = [ss0, ss1, ss2, ss3]
        gather(0, r0, gs0)
        gather(1, r1, gs1)

        def step(i, carry):
            for j in range(4):
                c4 = 4 * i + j
                wait_gather(bufs[j], gsems[j])
                scatter(bufs[j], c4, ssems[j])
                jn = (j + 2) % 4
                if j < 2:
                    @pl.when(i > 0)
                    def _():
                        wait_scatter(bufs[jn], ssems[jn])

                    gather(c4 + 2, bufs[jn], gsems[jn])
                else:
                    wait_scatter(bufs[jn], ssems[jn])

                    @pl.when(i < NCH // 4 - 1)
                    def _():
                        gather(c4 + 2, bufs[jn], gsems[jn])

            return carry

        lax.fori_loop(0, NCH // 4, step, 0)
        wait_scatter(r2, ss2)
        wait_scatter(r3, ss3)
        plsc.subcore_barrier()

    rows = pl.ds(s * ZR, ZR)
    for qq in range(NQ // NC):
        q = c * (NQ // NC) + qq
        cols = pl.ds(q * Q, Q)
        pltpu.sync_copy(xs_hbm.at[rows, cols], buf0.at[rows])
        pltpu.sync_copy(zeros_hbm, buf1.at[rows])
        plsc.subcore_barrier()
        run_pass(buf0, buf1)
        pltpu.sync_copy(buf1.at[rows], y1_hbm.at[rows, cols])
        pltpu.sync_copy(zeros_hbm, buf0.at[rows])
        plsc.subcore_barrier()
        run_pass(buf1, buf0)
        pltpu.sync_copy(buf0.at[rows], y2_hbm.at[rows, cols])
        plsc.subcore_barrier()


_dprop = pl.kernel(
    _dprop_body,
    out_type=(jax.ShapeDtypeStruct((NP, D), jnp.float32),
              jax.ShapeDtypeStruct((NP, D), jnp.float32)),
    mesh=plsc.VectorSubcoreMesh(core_axis_name="c", subcore_axis_name="s",
                                num_cores=NC, num_subcores=NS),
    scratch_types=[
        pltpu.VMEM((2 * NCH, CH), jnp.int32),
        pltpu.VMEM((CH, Q), jnp.float32),
        pltpu.VMEM((CH, Q), jnp.float32),
        pltpu.VMEM((CH, Q), jnp.float32),
        pltpu.VMEM((CH, Q), jnp.float32),
        pltpu.VMEM_SHARED((2, NP, Q), jnp.float32),
        pltpu.SemaphoreType.DMA,
        pltpu.SemaphoreType.DMA,
        pltpu.SemaphoreType.DMA,
        pltpu.SemaphoreType.DMA,
        pltpu.SemaphoreType.DMA,
        pltpu.SemaphoreType.DMA,
        pltpu.SemaphoreType.DMA,
        pltpu.SemaphoreType.DMA,
    ],
    compiler_params=pltpu.CompilerParams(use_tc_tiling_on_sc=False),
)


def _embed_body(h_ref, w_ref, b_ref, o_ref):
    x = (jnp.dot(h_ref[...], w_ref[...],
                 preferred_element_type=jnp.float32) + b_ref[...])
    o_ref[:N] = x


_embed = pl.pallas_call(
    _embed_body,
    out_shape=jax.ShapeDtypeStruct((NP, D), jnp.float32),
)


def _tail_body(x_ref, y1_ref, y2_ref, w_ref, b_ref, g_ref, bt_ref, o_ref):
    t = (jnp.dot(x_ref[:N], w_ref[0], preferred_element_type=jnp.float32)
         + jnp.dot(y1_ref[:N], w_ref[1], preferred_element_type=jnp.float32)
         + jnp.dot(y2_ref[:N], w_ref[2], preferred_element_type=jnp.float32)
         + b_ref[...])
    mu = jnp.mean(t, axis=0, keepdims=True)
    var = jnp.mean((t - mu) * (t - mu), axis=0, keepdims=True)
    t = (t - mu) * lax.rsqrt(var + 1e-5) * g_ref[...] + bt_ref[...]
    o_ref[:N] = jnp.maximum(t, 0.0) + x_ref[:N]


_tail = pl.pallas_call(
    _tail_body,
    out_shape=jax.ShapeDtypeStruct((NP, D), jnp.float32),
)


def kernel(h, e, edge_index, W_emb, b_emb, Wl, bl, gamma, beta):
    src = edge_index[0]
    dst = edge_index[1]
    pad = EPAD - E
    src_p = jnp.concatenate([src, jnp.zeros((pad,), jnp.int32)])
    dst_p = jnp.concatenate([dst, jnp.full((pad,), N, jnp.int32)])
    sidxm = src_p.reshape(NS, NCH, CH)
    didxm = dst_p.reshape(NS, NCH, CH)
    zeros = jnp.zeros((ZR, Q), jnp.float32)

    x = _embed(h, W_emb, b_emb.reshape(1, D))
    for l in range(L):
        y1, y2 = _dprop(sidxm, didxm, x, zeros)
        x = _tail(x, y1, y2, Wl[l],
                  (bl[l, 0] + bl[l, 1] + bl[l, 2]).reshape(1, D),
                  gamma[l].reshape(1, D), beta[l].reshape(1, D))
    return x[:N]
